# baseline (speedup 1.0000x reference)
import functools

import jax
import jax.numpy as jnp
from jax import lax
from jax.experimental import pallas as pl
from jax.experimental.pallas import tpu as pltpu
from jax.experimental.pallas import tpu_sc as plsc

_T, _D, _E, _H = 2048, 768, 8, 768
_P = 2 * _T
_BLK = 256
_NB = 24
_PP = _NB * _BLK
_NW = 32


def _route_body(x_ref, wg_ref, g0_ref, g1_ref, d0_ref, d1_ref, be_ref,
                loss_ref):
    x = x_ref[...]
    wg = wg_ref[...]
    logits = jnp.dot(x, wg, preferred_element_type=jnp.float32)
    eidx = lax.broadcasted_iota(jnp.int32, logits.shape, 1)
    m1 = jnp.max(logits, axis=1, keepdims=True)
    e1 = jnp.min(jnp.where(logits == m1, eidx, _E), axis=1, keepdims=True)
    oh1 = eidx == e1
    masked = jnp.where(oh1, -jnp.inf, logits)
    m2 = jnp.max(masked, axis=1, keepdims=True)
    e2 = jnp.min(jnp.where(masked == m2, eidx, _E), axis=1, keepdims=True)
    oh2 = eidx == e2
    b = jnp.exp(m2 - m1)
    ga = 1.0 / (1.0 + b)
    gb = b / (1.0 + b)
    g0_ref[...] = ga
    g1_ref[...] = gb
    gates = jnp.where(oh1, ga, 0.0) + jnp.where(oh2, gb, 0.0)
    imp = jnp.sum(gates, axis=0, keepdims=True)
    lod = jnp.sum((gates > 0).astype(jnp.float32), axis=0, keepdims=True)

    def cv_sq(v):
        m = jnp.sum(v) * (1.0 / _E)
        var = jnp.sum((v - m) ** 2) * (1.0 / _E)
        return var / (m * m + 1e-10)

    loss_ref[...] = jnp.full((1, 1), cv_sq(imp) + cv_sq(lod),
                             dtype=jnp.float32)

    oh_all = jnp.concatenate([oh1.astype(jnp.float32),
                              oh2.astype(jnp.float32)], axis=0)
    rtri = lax.broadcasted_iota(jnp.int32, (_BLK, _BLK), 0)
    ctri = lax.broadcasted_iota(jnp.int32, (_BLK, _BLK), 1)
    ltri = (rtri > ctri).astype(jnp.float32)
    ranks = []
    carry = jnp.zeros((1, _E), jnp.float32)
    for c in range(_P // _BLK):
        blk = lax.slice_in_dim(oh_all, c * _BLK, (c + 1) * _BLK, axis=0)
        ranks.append(jnp.dot(ltri, blk, preferred_element_type=jnp.float32)
                     + carry)
        carry = carry + jnp.sum(blk, axis=0, keepdims=True)
    rank = jnp.concatenate(ranks, axis=0)
    counts = carry
    nblk = jnp.floor((counts + (_BLK - 1)) * (1.0 / _BLK))
    er = lax.broadcasted_iota(jnp.int32, (_E, _E), 0)
    ec = lax.broadcasted_iota(jnp.int32, (_E, _E), 1)
    before = (er < ec).astype(jnp.float32)
    blkoff = jnp.dot(nblk, before, preferred_element_type=jnp.float32)
    padoff = _BLK * blkoff
    dest = jnp.sum((rank + padoff) * oh_all, axis=1, keepdims=True)
    d0_ref[...] = dest[:_T].astype(jnp.int32)
    d1_ref[...] = dest[_T:].astype(jnp.int32)
    cumblk = (blkoff + nblk).astype(jnp.int32)
    bi = lax.broadcasted_iota(jnp.int32, (_NB, _E), 0)
    be = jnp.sum((bi >= cumblk).astype(jnp.int32), axis=1)
    be_ref[...] = jnp.minimum(be, _E - 1).astype(jnp.int32)[None, :]


def _dispatch_body(x_hbm, d0_hbm, d1_hbm, xg_hbm, idx_v, rows_v, sem1):
    wid = lax.axis_index("s") * 2 + lax.axis_index("c")
    slot = wid // 16
    t0 = (wid % 16) * 128

    @pl.when(slot == 0)
    def _():
        pltpu.sync_copy(d0_hbm.at[pl.ds(t0, 128)], idx_v)

    @pl.when(slot == 1)
    def _():
        pltpu.sync_copy(d1_hbm.at[pl.ds(t0, 128)], idx_v)

    pltpu.sync_copy(x_hbm.at[pl.ds(t0, 128)], rows_v)
    pltpu.async_copy(rows_v, xg_hbm.at[idx_v], sem1).wait()


def _combine_body(yw_hbm, d0_hbm, d1_hbm, g0_hbm, g1_hbm, out_hbm,
                  i0_v, i1_v, g0_v, g1_v, r0_v, r1_v, sem0, sem1):
    wid = lax.axis_index("s") * 2 + lax.axis_index("c")
    t0 = wid * 64
    pltpu.sync_copy(d0_hbm.at[pl.ds(t0, 64)], i0_v)
    pltpu.sync_copy(d1_hbm.at[pl.ds(t0, 64)], i1_v)
    pltpu.sync_copy(g0_hbm.at[pl.ds(t0, 64)], g0_v.at[pl.ds(0, 64)])
    pltpu.sync_copy(g1_hbm.at[pl.ds(t0, 64)], g1_v.at[pl.ds(0, 64)])
    a = pltpu.async_copy(yw_hbm.at[i0_v], r0_v, sem0)
    b = pltpu.async_copy(yw_hbm.at[i1_v], r1_v, sem1)
    a.wait()
    b.wait()

    def row(i, _):
        g0 = g0_v[pl.ds(i, 16)][0]
        g1 = g1_v[pl.ds(i, 16)][0]
        for c in range(_D // 16):
            sl = pl.ds(c * 16, 16)
            r0_v[i, sl] = g0 * r0_v[i, sl] + g1 * r1_v[i, sl]
        return 0

    lax.fori_loop(0, 64, row, 0)
    pltpu.sync_copy(r0_v, out_hbm.at[pl.ds(t0, 64)])


def _ffn_body(be_ref, xg_ref, w1_ref, b1_ref, w2_ref, b2_ref, yw_ref):
    del be_ref
    xb = xg_ref[...].astype(jnp.bfloat16)
    h = jnp.dot(xb, w1_ref[0], preferred_element_type=jnp.float32)
    h = jnp.maximum(h + b1_ref[0], 0.0).astype(jnp.bfloat16)
    y = jnp.dot(h, w2_ref[0], preferred_element_type=jnp.float32)
    yw_ref[...] = y + b2_ref[0]


def kernel(x, w_gate, W1, b1, W2, b2):
    g0, g1, d0, d1, be, loss = pl.pallas_call(
        _route_body,
        grid=(1,),
        in_specs=[
            pl.BlockSpec((_T, _D), lambda i: (0, 0)),
            pl.BlockSpec((_D, _E), lambda i: (0, 0)),
        ],
        out_specs=[
            pl.BlockSpec((_T, 1), lambda i: (0, 0)),
            pl.BlockSpec((_T, 1), lambda i: (0, 0)),
            pl.BlockSpec((_T, 1), lambda i: (0, 0)),
            pl.BlockSpec((_T, 1), lambda i: (0, 0)),
            pl.BlockSpec((1, _NB), lambda i: (0, 0)),
            pl.BlockSpec((1, 1), lambda i: (0, 0)),
        ],
        out_shape=[
            jax.ShapeDtypeStruct((_T, 1), jnp.float32),
            jax.ShapeDtypeStruct((_T, 1), jnp.float32),
            jax.ShapeDtypeStruct((_T, 1), jnp.int32),
            jax.ShapeDtypeStruct((_T, 1), jnp.int32),
            jax.ShapeDtypeStruct((1, _NB), jnp.int32),
            jax.ShapeDtypeStruct((1, 1), jnp.float32),
        ],
    )(x, w_gate)

    d0f, d1f = d0.reshape(_T), d1.reshape(_T)
    g0f, g1f = g0.reshape(_T), g1.reshape(_T)

    mesh = plsc.VectorSubcoreMesh(core_axis_name="c", subcore_axis_name="s")

    dispatch = functools.partial(
        pl.kernel,
        mesh=mesh,
        out_type=jax.ShapeDtypeStruct((_PP, _D), jnp.float32),
        scratch_types=[
            pltpu.VMEM((128,), jnp.int32),
            pltpu.VMEM((128, _D), jnp.float32),
            pltpu.SemaphoreType.DMA,
        ],
    )(_dispatch_body)
    xg = dispatch(x, d0f, d1f)

    grid_spec = pltpu.PrefetchScalarGridSpec(
        num_scalar_prefetch=1,
        grid=(_NB,),
        in_specs=[
            pl.BlockSpec((_BLK, _D), lambda i, be: (i, 0)),
            pl.BlockSpec((1, _D, _H), lambda i, be: (be[i], 0, 0)),
            pl.BlockSpec((1, 1, _H), lambda i, be: (be[i], 0, 0)),
            pl.BlockSpec((1, _H, _D), lambda i, be: (be[i], 0, 0)),
            pl.BlockSpec((1, 1, _D), lambda i, be: (be[i], 0, 0)),
        ],
        out_specs=pl.BlockSpec((_BLK, _D), lambda i, be: (i, 0)),
    )
    yw = pl.pallas_call(
        _ffn_body,
        grid_spec=grid_spec,
        out_shape=jax.ShapeDtypeStruct((_PP, _D), jnp.float32),
    )(be.reshape(_NB), xg, W1.astype(jnp.bfloat16),
      b1.reshape(_E, 1, _H), W2.astype(jnp.bfloat16), b2.reshape(_E, 1, _D))

    combine = functools.partial(
        pl.kernel,
        mesh=mesh,
        out_type=jax.ShapeDtypeStruct((_T, _D), jnp.float32),
        scratch_types=[
            pltpu.VMEM((64,), jnp.int32),
            pltpu.VMEM((64,), jnp.int32),
            pltpu.VMEM((80,), jnp.float32),
            pltpu.VMEM((80,), jnp.float32),
            pltpu.VMEM((64, _D), jnp.float32),
            pltpu.VMEM((64, _D), jnp.float32),
            pltpu.SemaphoreType.DMA,
            pltpu.SemaphoreType.DMA,
        ],
    )(_combine_body)
    out = combine(yw, d0f, d1f, g0f, g1f)

    return out, loss[0, 0]


# speedup vs baseline: 1.1193x; 1.1193x over previous
import functools

import jax
import jax.numpy as jnp
from jax import lax
from jax.experimental import pallas as pl
from jax.experimental.pallas import tpu as pltpu
from jax.experimental.pallas import tpu_sc as plsc

_T, _D, _E, _H = 2048, 768, 8, 768
_P = 2 * _T
_BLK = 256
_NB = 24
_PP = _NB * _BLK
_NW = 32


def _route_body(x_ref, wg_ref, g0_ref, g1_ref, d0_ref, d1_ref, be_ref,
                loss_ref):
    x = x_ref[...]
    wg = wg_ref[...]
    logits = jnp.dot(x, wg, preferred_element_type=jnp.float32)
    eidx = lax.broadcasted_iota(jnp.int32, logits.shape, 1)
    m1 = jnp.max(logits, axis=1, keepdims=True)
    e1 = jnp.min(jnp.where(logits == m1, eidx, _E), axis=1, keepdims=True)
    oh1 = eidx == e1
    masked = jnp.where(oh1, -jnp.inf, logits)
    m2 = jnp.max(masked, axis=1, keepdims=True)
    e2 = jnp.min(jnp.where(masked == m2, eidx, _E), axis=1, keepdims=True)
    oh2 = eidx == e2
    b = jnp.exp(m2 - m1)
    ga = 1.0 / (1.0 + b)
    gb = b / (1.0 + b)
    g0_ref[...] = ga
    g1_ref[...] = gb
    gates = jnp.where(oh1, ga, 0.0) + jnp.where(oh2, gb, 0.0)
    imp = jnp.sum(gates, axis=0, keepdims=True)
    lod = jnp.sum((gates > 0).astype(jnp.float32), axis=0, keepdims=True)

    def cv_sq(v):
        m = jnp.sum(v) * (1.0 / _E)
        var = jnp.sum((v - m) ** 2) * (1.0 / _E)
        return var / (m * m + 1e-10)

    loss_ref[...] = jnp.full((1, 1), cv_sq(imp) + cv_sq(lod),
                             dtype=jnp.float32)

    oh_all = jnp.concatenate([oh1.astype(jnp.float32),
                              oh2.astype(jnp.float32)], axis=0)
    rtri = lax.broadcasted_iota(jnp.int32, (_BLK, _BLK), 0)
    ctri = lax.broadcasted_iota(jnp.int32, (_BLK, _BLK), 1)
    ltri = (rtri > ctri).astype(jnp.float32)
    ranks = []
    carry = jnp.zeros((1, _E), jnp.float32)
    for c in range(_P // _BLK):
        blk = lax.slice_in_dim(oh_all, c * _BLK, (c + 1) * _BLK, axis=0)
        ranks.append(jnp.dot(ltri, blk, preferred_element_type=jnp.float32)
                     + carry)
        carry = carry + jnp.sum(blk, axis=0, keepdims=True)
    rank = jnp.concatenate(ranks, axis=0)
    counts = carry
    nblk = jnp.floor((counts + (_BLK - 1)) * (1.0 / _BLK))
    er = lax.broadcasted_iota(jnp.int32, (_E, _E), 0)
    ec = lax.broadcasted_iota(jnp.int32, (_E, _E), 1)
    before = (er < ec).astype(jnp.float32)
    blkoff = jnp.dot(nblk, before, preferred_element_type=jnp.float32)
    padoff = _BLK * blkoff
    dest = jnp.sum((rank + padoff) * oh_all, axis=1, keepdims=True)
    d0_ref[...] = dest[:_T].astype(jnp.int32)
    d1_ref[...] = dest[_T:].astype(jnp.int32)
    cumblk = (blkoff + nblk).astype(jnp.int32)
    bi = lax.broadcasted_iota(jnp.int32, (_NB, _E), 0)
    be = jnp.sum((bi >= cumblk).astype(jnp.int32), axis=1)
    be_ref[...] = jnp.minimum(be, _E - 1).astype(jnp.int32)[None, :]


def _dispatch_body(x_hbm, d0_hbm, d1_hbm, xg_hbm, idx_v, rows_v, sem1):
    wid = lax.axis_index("s") * 2 + lax.axis_index("c")
    slot = wid // 16
    t0 = (wid % 16) * 128

    @pl.when(slot == 0)
    def _():
        pltpu.sync_copy(d0_hbm.at[pl.ds(t0, 128)], idx_v)

    @pl.when(slot == 1)
    def _():
        pltpu.sync_copy(d1_hbm.at[pl.ds(t0, 128)], idx_v)

    pltpu.sync_copy(x_hbm.at[pl.ds(t0, 128)], rows_v)
    pltpu.async_copy(rows_v, xg_hbm.at[idx_v], sem1).wait()


def _combine_body(yw_hbm, d0_hbm, d1_hbm, g0_hbm, g1_hbm, out_hbm,
                  i0_v, i1_v, g0_v, g1_v, r0_v, r1_v, sem0, sem1):
    wid = lax.axis_index("s") * 2 + lax.axis_index("c")
    t0 = wid * 64
    pltpu.sync_copy(d0_hbm.at[pl.ds(t0, 64)], i0_v)
    pltpu.sync_copy(d1_hbm.at[pl.ds(t0, 64)], i1_v)
    pltpu.sync_copy(g0_hbm.at[pl.ds(t0, 64)], g0_v.at[pl.ds(0, 64)])
    pltpu.sync_copy(g1_hbm.at[pl.ds(t0, 64)], g1_v.at[pl.ds(0, 64)])
    a = pltpu.async_copy(yw_hbm.at[i0_v], r0_v, sem0)
    b = pltpu.async_copy(yw_hbm.at[i1_v], r1_v, sem1)
    a.wait()
    b.wait()

    def row(i, _):
        g0 = g0_v[pl.ds(i, 16)][0]
        g1 = g1_v[pl.ds(i, 16)][0]
        for c in range(_D // 16):
            sl = pl.ds(c * 16, 16)
            r0_v[i, sl] = g0 * r0_v[i, sl] + g1 * r1_v[i, sl]
        return 0

    lax.fori_loop(0, 64, row, 0)
    pltpu.sync_copy(r0_v, out_hbm.at[pl.ds(t0, 64)])


def _ffn_body(be_ref, xg_ref, w1_ref, b1_ref, w2_ref, b2_ref, yw_ref):
    del be_ref
    h = jnp.dot(xg_ref[...], w1_ref[0], preferred_element_type=jnp.float32)
    h = jnp.maximum(h + b1_ref[0], 0.0)
    y = jnp.dot(h, w2_ref[0], preferred_element_type=jnp.float32)
    yw_ref[...] = y + b2_ref[0]


def kernel(x, w_gate, W1, b1, W2, b2):
    g0, g1, d0, d1, be, loss = pl.pallas_call(
        _route_body,
        grid=(1,),
        in_specs=[
            pl.BlockSpec((_T, _D), lambda i: (0, 0)),
            pl.BlockSpec((_D, _E), lambda i: (0, 0)),
        ],
        out_specs=[
            pl.BlockSpec((_T, 1), lambda i: (0, 0)),
            pl.BlockSpec((_T, 1), lambda i: (0, 0)),
            pl.BlockSpec((_T, 1), lambda i: (0, 0)),
            pl.BlockSpec((_T, 1), lambda i: (0, 0)),
            pl.BlockSpec((1, _NB), lambda i: (0, 0)),
            pl.BlockSpec((1, 1), lambda i: (0, 0)),
        ],
        out_shape=[
            jax.ShapeDtypeStruct((_T, 1), jnp.float32),
            jax.ShapeDtypeStruct((_T, 1), jnp.float32),
            jax.ShapeDtypeStruct((_T, 1), jnp.int32),
            jax.ShapeDtypeStruct((_T, 1), jnp.int32),
            jax.ShapeDtypeStruct((1, _NB), jnp.int32),
            jax.ShapeDtypeStruct((1, 1), jnp.float32),
        ],
    )(x, w_gate)

    d0f, d1f = d0.reshape(_T), d1.reshape(_T)
    g0f, g1f = g0.reshape(_T), g1.reshape(_T)

    mesh = plsc.VectorSubcoreMesh(core_axis_name="c", subcore_axis_name="s")

    dispatch = functools.partial(
        pl.kernel,
        mesh=mesh,
        out_type=jax.ShapeDtypeStruct((_PP, _D), jnp.float32),
        scratch_types=[
            pltpu.VMEM((128,), jnp.int32),
            pltpu.VMEM((128, _D), jnp.float32),
            pltpu.SemaphoreType.DMA,
        ],
    )(_dispatch_body)
    xg = dispatch(x, d0f, d1f)

    grid_spec = pltpu.PrefetchScalarGridSpec(
        num_scalar_prefetch=1,
        grid=(_NB,),
        in_specs=[
            pl.BlockSpec((_BLK, _D), lambda i, be: (i, 0)),
            pl.BlockSpec((1, _D, _H), lambda i, be: (be[i], 0, 0)),
            pl.BlockSpec((1, 1, _H), lambda i, be: (be[i], 0, 0)),
            pl.BlockSpec((1, _H, _D), lambda i, be: (be[i], 0, 0)),
            pl.BlockSpec((1, 1, _D), lambda i, be: (be[i], 0, 0)),
        ],
        out_specs=pl.BlockSpec((_BLK, _D), lambda i, be: (i, 0)),
    )
    yw = pl.pallas_call(
        _ffn_body,
        grid_spec=grid_spec,
        out_shape=jax.ShapeDtypeStruct((_PP, _D), jnp.float32),
    )(be.reshape(_NB), xg, W1,
      b1.reshape(_E, 1, _H), W2, b2.reshape(_E, 1, _D))

    combine = functools.partial(
        pl.kernel,
        mesh=mesh,
        out_type=jax.ShapeDtypeStruct((_T, _D), jnp.float32),
        scratch_types=[
            pltpu.VMEM((64,), jnp.int32),
            pltpu.VMEM((64,), jnp.int32),
            pltpu.VMEM((80,), jnp.float32),
            pltpu.VMEM((80,), jnp.float32),
            pltpu.VMEM((64, _D), jnp.float32),
            pltpu.VMEM((64, _D), jnp.float32),
            pltpu.SemaphoreType.DMA,
            pltpu.SemaphoreType.DMA,
        ],
    )(_combine_body)
    out = combine(yw, d0f, d1f, g0f, g1f)

    return out, loss[0, 0]


# speedup vs baseline: 1.1311x; 1.0106x over previous
import functools

import jax
import jax.numpy as jnp
from jax import lax
from jax.experimental import pallas as pl
from jax.experimental.pallas import tpu as pltpu
from jax.experimental.pallas import tpu_sc as plsc

_T, _D, _E, _H = 2048, 768, 8, 768
_P = 2 * _T
_BLK = 256
_NB = 24
_PP = _NB * _BLK
_NW = 32


def _route_body(x_ref, wg_ref, g0_ref, g1_ref, d0_ref, d1_ref, be_ref,
                loss_ref):
    x = x_ref[...]
    wg = wg_ref[...]
    logits = jnp.dot(x, wg, preferred_element_type=jnp.float32)
    eidx = lax.broadcasted_iota(jnp.int32, logits.shape, 1)
    m1 = jnp.max(logits, axis=1, keepdims=True)
    e1 = jnp.min(jnp.where(logits == m1, eidx, _E), axis=1, keepdims=True)
    oh1 = eidx == e1
    masked = jnp.where(oh1, -jnp.inf, logits)
    m2 = jnp.max(masked, axis=1, keepdims=True)
    e2 = jnp.min(jnp.where(masked == m2, eidx, _E), axis=1, keepdims=True)
    oh2 = eidx == e2
    b = jnp.exp(m2 - m1)
    ga = 1.0 / (1.0 + b)
    gb = b / (1.0 + b)
    g0_ref[...] = ga
    g1_ref[...] = gb
    gates = jnp.where(oh1, ga, 0.0) + jnp.where(oh2, gb, 0.0)
    imp = jnp.sum(gates, axis=0, keepdims=True)
    lod = jnp.sum((gates > 0).astype(jnp.float32), axis=0, keepdims=True)

    def cv_sq(v):
        m = jnp.sum(v) * (1.0 / _E)
        var = jnp.sum((v - m) ** 2) * (1.0 / _E)
        return var / (m * m + 1e-10)

    loss_ref[...] = jnp.full((1, 1), cv_sq(imp) + cv_sq(lod),
                             dtype=jnp.float32)

    oh_all = jnp.concatenate([oh1.astype(jnp.float32),
                              oh2.astype(jnp.float32)], axis=0)
    rtri = lax.broadcasted_iota(jnp.int32, (_BLK, _BLK), 0)
    ctri = lax.broadcasted_iota(jnp.int32, (_BLK, _BLK), 1)
    ltri = (rtri > ctri).astype(jnp.float32)
    ranks = []
    carry = jnp.zeros((1, _E), jnp.float32)
    for c in range(_P // _BLK):
        blk = lax.slice_in_dim(oh_all, c * _BLK, (c + 1) * _BLK, axis=0)
        ranks.append(jnp.dot(ltri, blk, preferred_element_type=jnp.float32)
                     + carry)
        carry = carry + jnp.sum(blk, axis=0, keepdims=True)
    rank = jnp.concatenate(ranks, axis=0)
    counts = carry
    nblk = jnp.floor((counts + (_BLK - 1)) * (1.0 / _BLK))
    er = lax.broadcasted_iota(jnp.int32, (_E, _E), 0)
    ec = lax.broadcasted_iota(jnp.int32, (_E, _E), 1)
    before = (er < ec).astype(jnp.float32)
    blkoff = jnp.dot(nblk, before, preferred_element_type=jnp.float32)
    padoff = _BLK * blkoff
    dest = jnp.sum((rank + padoff) * oh_all, axis=1, keepdims=True)
    d0_ref[...] = dest[:_T].astype(jnp.int32)
    d1_ref[...] = dest[_T:].astype(jnp.int32)
    cumblk = (blkoff + nblk).astype(jnp.int32)
    bi = lax.broadcasted_iota(jnp.int32, (_NB, _E), 0)
    be = jnp.sum((bi >= cumblk).astype(jnp.int32), axis=1)
    be_ref[...] = be.astype(jnp.int32)[None, :]


def _dispatch_body(x_hbm, d0_hbm, d1_hbm, xg_hbm, idx_v, rows_v, sem1):
    wid = lax.axis_index("s") * 2 + lax.axis_index("c")
    slot = wid // 16
    t0 = (wid % 16) * 128

    @pl.when(slot == 0)
    def _():
        pltpu.sync_copy(d0_hbm.at[pl.ds(t0, 128)], idx_v)

    @pl.when(slot == 1)
    def _():
        pltpu.sync_copy(d1_hbm.at[pl.ds(t0, 128)], idx_v)

    pltpu.sync_copy(x_hbm.at[pl.ds(t0, 128)], rows_v)
    pltpu.async_copy(rows_v, xg_hbm.at[idx_v], sem1).wait()


def _combine_body(yw_hbm, d0_hbm, d1_hbm, g0_hbm, g1_hbm, out_hbm,
                  i0_v, i1_v, g0_v, g1_v, r0_v, r1_v, sem0, sem1,
                  sem2, sem3):
    wid = lax.axis_index("s") * 2 + lax.axis_index("c")
    t0 = wid * 64
    pltpu.sync_copy(d0_hbm.at[pl.ds(t0, 64)], i0_v)
    pltpu.sync_copy(d1_hbm.at[pl.ds(t0, 64)], i1_v)
    pltpu.sync_copy(g0_hbm.at[pl.ds(t0, 64)], g0_v.at[pl.ds(0, 64)])
    pltpu.sync_copy(g1_hbm.at[pl.ds(t0, 64)], g1_v.at[pl.ds(0, 64)])
    cps = []
    for half, sems in ((0, (sem0, sem1)), (1, (sem2, sem3))):
        rs = pl.ds(half * 32, 32)
        cps.append(pltpu.async_copy(yw_hbm.at[i0_v.at[rs]], r0_v.at[rs],
                                    sems[0]))
        cps.append(pltpu.async_copy(yw_hbm.at[i1_v.at[rs]], r1_v.at[rs],
                                    sems[1]))

    def row(i, _):
        g0 = g0_v[pl.ds(i, 16)][0]
        g1 = g1_v[pl.ds(i, 16)][0]
        for c in range(_D // 16):
            sl = pl.ds(c * 16, 16)
            r0_v[i, sl] = g0 * r0_v[i, sl] + g1 * r1_v[i, sl]
        return 0

    cps[0].wait()
    cps[1].wait()
    lax.fori_loop(0, 32, row, 0)
    cps[2].wait()
    cps[3].wait()
    lax.fori_loop(32, 64, row, 0)
    pltpu.sync_copy(r0_v, out_hbm.at[pl.ds(t0, 64)])


def _ffn_body(be_ref, xg_ref, w1_ref, b1_ref, w2_ref, b2_ref, yw_ref):
    i = pl.program_id(0)

    @pl.when(be_ref[i] < _E)
    def _():
        h = jnp.dot(xg_ref[...], w1_ref[0],
                    preferred_element_type=jnp.float32)
        h = jnp.maximum(h + b1_ref[0], 0.0)
        y = jnp.dot(h, w2_ref[0], preferred_element_type=jnp.float32)
        yw_ref[...] = y + b2_ref[0]


def kernel(x, w_gate, W1, b1, W2, b2):
    g0, g1, d0, d1, be, loss = pl.pallas_call(
        _route_body,
        grid=(1,),
        in_specs=[
            pl.BlockSpec((_T, _D), lambda i: (0, 0)),
            pl.BlockSpec((_D, _E), lambda i: (0, 0)),
        ],
        out_specs=[
            pl.BlockSpec((_T, 1), lambda i: (0, 0)),
            pl.BlockSpec((_T, 1), lambda i: (0, 0)),
            pl.BlockSpec((_T, 1), lambda i: (0, 0)),
            pl.BlockSpec((_T, 1), lambda i: (0, 0)),
            pl.BlockSpec((1, _NB), lambda i: (0, 0)),
            pl.BlockSpec((1, 1), lambda i: (0, 0)),
        ],
        out_shape=[
            jax.ShapeDtypeStruct((_T, 1), jnp.float32),
            jax.ShapeDtypeStruct((_T, 1), jnp.float32),
            jax.ShapeDtypeStruct((_T, 1), jnp.int32),
            jax.ShapeDtypeStruct((_T, 1), jnp.int32),
            jax.ShapeDtypeStruct((1, _NB), jnp.int32),
            jax.ShapeDtypeStruct((1, 1), jnp.float32),
        ],
    )(x, w_gate)

    d0f, d1f = d0.reshape(_T), d1.reshape(_T)
    g0f, g1f = g0.reshape(_T), g1.reshape(_T)

    mesh = plsc.VectorSubcoreMesh(core_axis_name="c", subcore_axis_name="s")

    dispatch = functools.partial(
        pl.kernel,
        mesh=mesh,
        out_type=jax.ShapeDtypeStruct((_PP, _D), jnp.float32),
        scratch_types=[
            pltpu.VMEM((128,), jnp.int32),
            pltpu.VMEM((128, _D), jnp.float32),
            pltpu.SemaphoreType.DMA,
        ],
    )(_dispatch_body)
    xg = dispatch(x, d0f, d1f)

    grid_spec = pltpu.PrefetchScalarGridSpec(
        num_scalar_prefetch=1,
        grid=(_NB,),
        in_specs=[
            pl.BlockSpec((_BLK, _D), lambda i, be: (i, 0)),
            pl.BlockSpec((1, _D, _H),
                         lambda i, be: (jnp.minimum(be[i], _E - 1), 0, 0)),
            pl.BlockSpec((1, 1, _H),
                         lambda i, be: (jnp.minimum(be[i], _E - 1), 0, 0)),
            pl.BlockSpec((1, _H, _D),
                         lambda i, be: (jnp.minimum(be[i], _E - 1), 0, 0)),
            pl.BlockSpec((1, 1, _D),
                         lambda i, be: (jnp.minimum(be[i], _E - 1), 0, 0)),
        ],
        out_specs=pl.BlockSpec((_BLK, _D), lambda i, be: (i, 0)),
    )
    yw = pl.pallas_call(
        _ffn_body,
        grid_spec=grid_spec,
        out_shape=jax.ShapeDtypeStruct((_PP, _D), jnp.float32),
    )(be.reshape(_NB), xg, W1,
      b1.reshape(_E, 1, _H), W2, b2.reshape(_E, 1, _D))

    combine = functools.partial(
        pl.kernel,
        mesh=mesh,
        out_type=jax.ShapeDtypeStruct((_T, _D), jnp.float32),
        scratch_types=[
            pltpu.VMEM((64,), jnp.int32),
            pltpu.VMEM((64,), jnp.int32),
            pltpu.VMEM((80,), jnp.float32),
            pltpu.VMEM((80,), jnp.float32),
            pltpu.VMEM((64, _D), jnp.float32),
            pltpu.VMEM((64, _D), jnp.float32),
            pltpu.SemaphoreType.DMA,
            pltpu.SemaphoreType.DMA,
            pltpu.SemaphoreType.DMA,
            pltpu.SemaphoreType.DMA,
        ],
    )(_combine_body)
    out = combine(yw, d0f, d1f, g0f, g1f)

    return out, loss[0, 0]


# speedup vs baseline: 1.2147x; 1.0738x over previous
import functools

import jax
import jax.numpy as jnp
from jax import lax
from jax.experimental import pallas as pl
from jax.experimental.pallas import tpu as pltpu
from jax.experimental.pallas import tpu_sc as plsc

_T, _D, _E, _H = 2048, 768, 8, 768
_P = 2 * _T
_BLK = 512
_NB = 15
_PP = _NB * _BLK
_NW = 32


def _route_body(x_ref, wg_ref, g0_ref, g1_ref, d0_ref, d1_ref, be_ref,
                loss_ref):
    x = x_ref[...]
    wg = wg_ref[...]
    logits = jnp.dot(x, wg, preferred_element_type=jnp.float32)
    eidx = lax.broadcasted_iota(jnp.int32, logits.shape, 1)
    m1 = jnp.max(logits, axis=1, keepdims=True)
    e1 = jnp.min(jnp.where(logits == m1, eidx, _E), axis=1, keepdims=True)
    oh1 = eidx == e1
    masked = jnp.where(oh1, -jnp.inf, logits)
    m2 = jnp.max(masked, axis=1, keepdims=True)
    e2 = jnp.min(jnp.where(masked == m2, eidx, _E), axis=1, keepdims=True)
    oh2 = eidx == e2
    b = jnp.exp(m2 - m1)
    ga = 1.0 / (1.0 + b)
    gb = b / (1.0 + b)
    g0_ref[...] = ga
    g1_ref[...] = gb
    gates = jnp.where(oh1, ga, 0.0) + jnp.where(oh2, gb, 0.0)
    imp = jnp.sum(gates, axis=0, keepdims=True)
    lod = jnp.sum((gates > 0).astype(jnp.float32), axis=0, keepdims=True)

    def cv_sq(v):
        m = jnp.sum(v) * (1.0 / _E)
        var = jnp.sum((v - m) ** 2) * (1.0 / _E)
        return var / (m * m + 1e-10)

    loss_ref[...] = jnp.full((1, 1), cv_sq(imp) + cv_sq(lod),
                             dtype=jnp.float32)

    oh_all = jnp.concatenate([oh1.astype(jnp.float32),
                              oh2.astype(jnp.float32)], axis=0)
    rtri = lax.broadcasted_iota(jnp.int32, (_BLK, _BLK), 0)
    ctri = lax.broadcasted_iota(jnp.int32, (_BLK, _BLK), 1)
    ltri = (rtri > ctri).astype(jnp.float32)
    ranks = []
    carry = jnp.zeros((1, _E), jnp.float32)
    for c in range(_P // _BLK):
        blk = lax.slice_in_dim(oh_all, c * _BLK, (c + 1) * _BLK, axis=0)
        ranks.append(jnp.dot(ltri, blk, preferred_element_type=jnp.float32)
                     + carry)
        carry = carry + jnp.sum(blk, axis=0, keepdims=True)
    rank = jnp.concatenate(ranks, axis=0)
    counts = carry
    nblk = jnp.floor((counts + (_BLK - 1)) * (1.0 / _BLK))
    er = lax.broadcasted_iota(jnp.int32, (_E, _E), 0)
    ec = lax.broadcasted_iota(jnp.int32, (_E, _E), 1)
    before = (er < ec).astype(jnp.float32)
    blkoff = jnp.dot(nblk, before, preferred_element_type=jnp.float32)
    padoff = _BLK * blkoff
    dest = jnp.sum((rank + padoff) * oh_all, axis=1, keepdims=True)
    d0_ref[...] = dest[:_T].astype(jnp.int32)
    d1_ref[...] = dest[_T:].astype(jnp.int32)
    cumblk = (blkoff + nblk).astype(jnp.int32)
    bi = lax.broadcasted_iota(jnp.int32, (_NB, _E), 0)
    be = jnp.sum((bi >= cumblk).astype(jnp.int32), axis=1)
    be_ref[...] = be.astype(jnp.int32)[None, :]


def _dispatch_body(x_hbm, d0_hbm, d1_hbm, xg_hbm, idx_v, rows_v, sem1):
    wid = lax.axis_index("s") * 2 + lax.axis_index("c")
    slot = wid // 16
    t0 = (wid % 16) * 128

    @pl.when(slot == 0)
    def _():
        pltpu.sync_copy(d0_hbm.at[pl.ds(t0, 128)], idx_v)

    @pl.when(slot == 1)
    def _():
        pltpu.sync_copy(d1_hbm.at[pl.ds(t0, 128)], idx_v)

    pltpu.sync_copy(x_hbm.at[pl.ds(t0, 128)], rows_v)
    pltpu.async_copy(rows_v, xg_hbm.at[idx_v], sem1).wait()


def _combine_body(yw_hbm, d0_hbm, d1_hbm, g0_hbm, g1_hbm, out_hbm,
                  i0_v, i1_v, g0_v, g1_v, r0_v, r1_v, sem0, sem1,
                  sem2, sem3):
    wid = lax.axis_index("s") * 2 + lax.axis_index("c")
    t0 = wid * 64
    pltpu.sync_copy(d0_hbm.at[pl.ds(t0, 64)], i0_v)
    pltpu.sync_copy(d1_hbm.at[pl.ds(t0, 64)], i1_v)
    pltpu.sync_copy(g0_hbm.at[pl.ds(t0, 64)], g0_v.at[pl.ds(0, 64)])
    pltpu.sync_copy(g1_hbm.at[pl.ds(t0, 64)], g1_v.at[pl.ds(0, 64)])
    cps = []
    for half, sems in ((0, (sem0, sem1)), (1, (sem2, sem3))):
        rs = pl.ds(half * 32, 32)
        cps.append(pltpu.async_copy(yw_hbm.at[i0_v.at[rs]], r0_v.at[rs],
                                    sems[0]))
        cps.append(pltpu.async_copy(yw_hbm.at[i1_v.at[rs]], r1_v.at[rs],
                                    sems[1]))

    def row(i, _):
        g0 = g0_v[pl.ds(i, 16)][0]
        g1 = g1_v[pl.ds(i, 16)][0]
        for c in range(_D // 16):
            sl = pl.ds(c * 16, 16)
            r0_v[i, sl] = g0 * r0_v[i, sl] + g1 * r1_v[i, sl]
        return 0

    cps[0].wait()
    cps[1].wait()
    lax.fori_loop(0, 32, row, 0)
    cps[2].wait()
    cps[3].wait()
    lax.fori_loop(32, 64, row, 0)
    pltpu.sync_copy(r0_v, out_hbm.at[pl.ds(t0, 64)])


def _ffn_body(be_ref, xg_ref, w1_ref, b1_ref, w2_ref, b2_ref, yw_ref):
    i = pl.program_id(0)

    @pl.when(be_ref[i] < _E)
    def _():
        h = jnp.dot(xg_ref[...], w1_ref[0],
                    preferred_element_type=jnp.float32)
        h = jnp.maximum(h + b1_ref[0], 0.0)
        y = jnp.dot(h, w2_ref[0], preferred_element_type=jnp.float32)
        yw_ref[...] = y + b2_ref[0]


def kernel(x, w_gate, W1, b1, W2, b2):
    g0, g1, d0, d1, be, loss = pl.pallas_call(
        _route_body,
        grid=(1,),
        in_specs=[
            pl.BlockSpec((_T, _D), lambda i: (0, 0)),
            pl.BlockSpec((_D, _E), lambda i: (0, 0)),
        ],
        out_specs=[
            pl.BlockSpec((_T, 1), lambda i: (0, 0)),
            pl.BlockSpec((_T, 1), lambda i: (0, 0)),
            pl.BlockSpec((_T, 1), lambda i: (0, 0)),
            pl.BlockSpec((_T, 1), lambda i: (0, 0)),
            pl.BlockSpec((1, _NB), lambda i: (0, 0)),
            pl.BlockSpec((1, 1), lambda i: (0, 0)),
        ],
        out_shape=[
            jax.ShapeDtypeStruct((_T, 1), jnp.float32),
            jax.ShapeDtypeStruct((_T, 1), jnp.float32),
            jax.ShapeDtypeStruct((_T, 1), jnp.int32),
            jax.ShapeDtypeStruct((_T, 1), jnp.int32),
            jax.ShapeDtypeStruct((1, _NB), jnp.int32),
            jax.ShapeDtypeStruct((1, 1), jnp.float32),
        ],
    )(x, w_gate)

    d0f, d1f = d0.reshape(_T), d1.reshape(_T)
    g0f, g1f = g0.reshape(_T), g1.reshape(_T)

    mesh = plsc.VectorSubcoreMesh(core_axis_name="c", subcore_axis_name="s")

    dispatch = functools.partial(
        pl.kernel,
        mesh=mesh,
        out_type=jax.ShapeDtypeStruct((_PP, _D), jnp.float32),
        scratch_types=[
            pltpu.VMEM((128,), jnp.int32),
            pltpu.VMEM((128, _D), jnp.float32),
            pltpu.SemaphoreType.DMA,
        ],
    )(_dispatch_body)
    xg = dispatch(x, d0f, d1f)

    grid_spec = pltpu.PrefetchScalarGridSpec(
        num_scalar_prefetch=1,
        grid=(_NB,),
        in_specs=[
            pl.BlockSpec((_BLK, _D), lambda i, be: (i, 0)),
            pl.BlockSpec((1, _D, _H),
                         lambda i, be: (jnp.minimum(be[i], _E - 1), 0, 0)),
            pl.BlockSpec((1, 1, _H),
                         lambda i, be: (jnp.minimum(be[i], _E - 1), 0, 0)),
            pl.BlockSpec((1, _H, _D),
                         lambda i, be: (jnp.minimum(be[i], _E - 1), 0, 0)),
            pl.BlockSpec((1, 1, _D),
                         lambda i, be: (jnp.minimum(be[i], _E - 1), 0, 0)),
        ],
        out_specs=pl.BlockSpec((_BLK, _D), lambda i, be: (i, 0)),
    )
    yw = pl.pallas_call(
        _ffn_body,
        grid_spec=grid_spec,
        out_shape=jax.ShapeDtypeStruct((_PP, _D), jnp.float32),
    )(be.reshape(_NB), xg, W1,
      b1.reshape(_E, 1, _H), W2, b2.reshape(_E, 1, _D))

    combine = functools.partial(
        pl.kernel,
        mesh=mesh,
        out_type=jax.ShapeDtypeStruct((_T, _D), jnp.float32),
        scratch_types=[
            pltpu.VMEM((64,), jnp.int32),
            pltpu.VMEM((64,), jnp.int32),
            pltpu.VMEM((80,), jnp.float32),
            pltpu.VMEM((80,), jnp.float32),
            pltpu.VMEM((64, _D), jnp.float32),
            pltpu.VMEM((64, _D), jnp.float32),
            pltpu.SemaphoreType.DMA,
            pltpu.SemaphoreType.DMA,
            pltpu.SemaphoreType.DMA,
            pltpu.SemaphoreType.DMA,
        ],
    )(_combine_body)
    out = combine(yw, d0f, d1f, g0f, g1f)

    return out, loss[0, 0]
